# baseline (speedup 1.0000x reference)
import functools

import jax
import jax.numpy as jnp
from jax import lax
from jax.experimental import pallas as pl
from jax.experimental.pallas import tpu as pltpu
from jax.experimental.pallas import tpu_sc as plsc

_N = 10000
_NP = 10240
_E = 320000
_F = 128
_NCLS = 40

_NC = 2
_NS = 16
_NW = _NC * _NS
_CHUNK = 128
_NCHP = 2560
_TCH = _NCHP // _NW
_IBLK = 16
_EPAD = _NCHP * _CHUNK - _E
_ZROWS = _NP // _NS


def _zero_acc(rows_v, acc, r0, z_hbm, step, sem):
    pltpu.sync_copy(z_hbm, rows_v)

    def zbody(k, carry):
        pltpu.async_copy(rows_v, acc.at[pl.ds(r0 + k * step, step)], sem)
        return carry

    lax.fori_loop(0, _ZROWS // step, zbody, 0)

    def zwait(k, carry):
        pltpu.make_async_copy(rows_v, acc.at[pl.ds(r0 + k * step, step)],
                              sem).wait()
        return carry

    lax.fori_loop(0, _ZROWS // step, zwait, 0)


def _flush_acc(rows_v, acc, r0, cid, out_hbm, step):
    def fbody(k, carry):
        rr = r0 + k * step
        pltpu.sync_copy(acc.at[pl.ds(rr, step)], rows_v)
        pltpu.sync_copy(rows_v, out_hbm.at[cid, pl.ds(rr, step)])
        return carry

    lax.fori_loop(0, _ZROWS // step, fbody, 0)


_DEPTH = 4
_GCH = 64
_TSC = _NCHP * _CHUNK // _GCH // _NW
_PH = _TSC // 2


def _sc_agg_body(h_hbm, src_hbm, dst_hbm, z_hbm, out_hbm, *scr):
    src2_v = scr[0]
    dsts = scr[1:1 + _DEPTH]
    rows = scr[1 + _DEPTH:1 + 2 * _DEPTH]
    acc = scr[1 + 2 * _DEPTH]
    gsems = scr[2 + 2 * _DEPTH:2 + 3 * _DEPTH]
    dsems = scr[2 + 3 * _DEPTH:2 + 4 * _DEPTH]
    cid = lax.axis_index("c")
    sid = lax.axis_index("s")
    wid = sid * _NC + cid
    r0 = sid * _ZROWS
    _zero_acc(rows[0], acc, r0, z_hbm, _GCH, gsems[0])

    c0 = wid * _TSC
    pltpu.sync_copy(src_hbm.at[pl.ds(c0, _PH)], src2_v)
    plsc.subcore_barrier()

    def run_pipe(gb):
        def dst_ref(j):
            return dst_hbm.at[pl.ds((c0 + gb + j) * _GCH, _GCH)]

        def fire(j, p):
            pltpu.async_copy(dst_ref(j), dsts[p], dsems[p])
            pltpu.async_copy(h_hbm.at[src2_v.at[j]], rows[p], gsems[p])

        for p in range(_DEPTH):
            fire(p, p)

        def body(t, carry):
            for p in range(_DEPTH):
                j = _DEPTH * t + p
                pltpu.make_async_copy(dst_ref(j), dsts[p], dsems[p]).wait()
                pltpu.make_async_copy(h_hbm.at[src2_v.at[j]], rows[p],
                                      gsems[p]).wait()
                pltpu.sync_copy(rows[p], acc.at[dsts[p]], add=True)

                @pl.when(t < _PH // _DEPTH - 1)
                def _():
                    fire(j + _DEPTH, p)
            return carry

        lax.fori_loop(0, _PH // _DEPTH, body, 0)

    run_pipe(0)
    pltpu.sync_copy(src_hbm.at[pl.ds(c0 + _PH, _PH)], src2_v)
    run_pipe(_PH)

    plsc.subcore_barrier()
    _flush_acc(rows[0], acc, r0, cid, out_hbm, _GCH)


def _sc_deg_body(dst_hbm, z_hbm, ones_hbm, out_hbm,
                 dst_v, rows_v, acc, sem):
    cid = lax.axis_index("c")
    sid = lax.axis_index("s")
    wid = sid * _NC + cid
    r0 = sid * _ZROWS
    _zero_acc(rows_v, acc, r0, z_hbm, _CHUNK, sem)
    pltpu.sync_copy(ones_hbm, rows_v)
    plsc.subcore_barrier()

    e0 = wid * _TCH * _CHUNK

    def body(j, carry):
        pltpu.sync_copy(dst_hbm.at[pl.ds(e0 + j * _CHUNK, _CHUNK)], dst_v)
        pltpu.sync_copy(rows_v, acc.at[dst_v], add=True)
        return carry

    lax.fori_loop(0, _TCH, body, 0)

    plsc.subcore_barrier()
    _flush_acc(rows_v, acc, r0, cid, out_hbm, _CHUNK)


_PLANES = jax.ShapeDtypeStruct((_NC, _NP, _F), jnp.float32)


@functools.lru_cache(maxsize=None)
def _make_sc_agg():
    mesh = plsc.VectorSubcoreMesh(core_axis_name="c", subcore_axis_name="s",
                                  num_cores=_NC, num_subcores=_NS)
    return pl.kernel(
        _sc_agg_body,
        out_type=_PLANES,
        mesh=mesh,
        scratch_types=(
            [pltpu.VMEM((_PH, _GCH), jnp.int32)]
            + [pltpu.VMEM((_GCH,), jnp.int32)] * _DEPTH
            + [pltpu.VMEM((_GCH, _F), jnp.float32)] * _DEPTH
            + [pltpu.VMEM_SHARED((_NP, _F), jnp.float32)]
            + [pltpu.SemaphoreType.DMA] * (2 * _DEPTH)
        ),
    )


@functools.lru_cache(maxsize=None)
def _make_sc_deg():
    mesh = plsc.VectorSubcoreMesh(core_axis_name="c", subcore_axis_name="s",
                                  num_cores=_NC, num_subcores=_NS)
    return pl.kernel(
        _sc_deg_body,
        out_type=_PLANES,
        mesh=mesh,
        scratch_types=[
            pltpu.VMEM((_CHUNK,), jnp.int32),
            pltpu.VMEM((_CHUNK, _F), jnp.float32),
            pltpu.VMEM_SHARED((_NP, _F), jnp.float32),
            pltpu.SemaphoreType.DMA,
        ],
    )


def _deg_body(d_ref, o_ref):
    o_ref[...] = 1.0 / jnp.maximum(d_ref[0] + d_ref[1], 1.0)


_deg_prep = pl.pallas_call(
    _deg_body,
    grid=(_NP // 2048,),
    in_specs=[pl.BlockSpec((_NC, 2048, _F), lambda r: (0, r, 0))],
    out_specs=pl.BlockSpec((2048, _F), lambda r: (r, 0)),
    out_shape=jax.ShapeDtypeStruct((_NP, _F), jnp.float32),
)

_BLK = 2048


def _layer_body(i, act, eps_ref, h_ref, p_ref, inv_ref, w_ref, b_ref, o_ref):
    agg = (p_ref[0] + p_ref[1]) * inv_ref[...]
    rst = (1.0 + eps_ref[i]) * h_ref[...] + agg
    o = jnp.dot(rst, w_ref[...], preferred_element_type=jnp.float32) + b_ref[...]
    o_ref[...] = jnp.maximum(o, 0.0) if act else o


def _make_layer(i, act):
    return pl.pallas_call(
        functools.partial(_layer_body, i, act),
        grid=(_NP // _BLK,),
        in_specs=[
            pl.BlockSpec(memory_space=pltpu.SMEM),
            pl.BlockSpec((_BLK, _F), lambda r: (r, 0)),
            pl.BlockSpec((_NC, _BLK, _F), lambda r: (0, r, 0)),
            pl.BlockSpec((_BLK, _F), lambda r: (r, 0)),
            pl.BlockSpec((_F, _F), lambda r: (0, 0)),
            pl.BlockSpec((1, _F), lambda r: (0, 0)),
        ],
        out_specs=pl.BlockSpec((_BLK, _F), lambda r: (r, 0)),
        out_shape=jax.ShapeDtypeStruct((_NP, _F), jnp.float32),
    )


_layers = [_make_layer(0, True), _make_layer(1, True),
           _make_layer(2, True), _make_layer(3, False)]


def kernel(features, edge_index, W0, b0, W1, b1, W2, b2, W3, b3, eps):
    h = jnp.zeros((_NP, _F), jnp.float32).at[:_N].set(features)
    src_pad = jnp.zeros((_EPAD,), jnp.int32)
    dst_pad = _N + (jnp.arange(_EPAD, dtype=jnp.int32) % (_NP - _N))
    src1 = jnp.concatenate([edge_index[0], src_pad])
    dst1 = jnp.concatenate([edge_index[1], dst_pad])
    zeros = jnp.zeros((_CHUNK, _F), jnp.float32)
    ones = jnp.ones((_CHUNK, _F), jnp.float32)

    W3p = jnp.pad(W3, ((0, 0), (0, _F - _NCLS)))
    b3p = jnp.pad(b3, (0, _F - _NCLS))
    ws = [(W0, b0), (W1, b1), (W2, b2), (W3p, b3p)]

    src2d = src1.reshape(-1, _GCH)
    zeros64 = jnp.zeros((_GCH, _F), jnp.float32)
    dplanes = _make_sc_deg()(dst1, zeros, ones)
    inv = _deg_prep(dplanes)
    for i in range(4):
        planes = _make_sc_agg()(h, src2d, dst1, zeros64)
        W, b = ws[i]
        h = _layers[i](eps, h, planes, inv, W, b.reshape(1, _F))
    return h[:_N, :_NCLS]


# speedup vs baseline: 1.0006x; 1.0006x over previous
import functools

import jax
import jax.numpy as jnp
from jax import lax
from jax.experimental import pallas as pl
from jax.experimental.pallas import tpu as pltpu
from jax.experimental.pallas import tpu_sc as plsc

_N = 10000
_NP = 10240
_E = 320000
_F = 128
_NCLS = 40

_NC = 2
_NS = 16
_NW = _NC * _NS
_CHUNK = 128
_NCHP = 2560
_TCH = _NCHP // _NW
_EPAD = _NCHP * _CHUNK - _E
_ZROWS = _NP // _NS


def _zero_acc(rows_v, acc, r0, z_hbm, step, sem):
    pltpu.sync_copy(z_hbm, rows_v)

    def zbody(k, carry):
        pltpu.async_copy(rows_v, acc.at[pl.ds(r0 + k * step, step)], sem)
        return carry

    lax.fori_loop(0, _ZROWS // step, zbody, 0)

    def zwait(k, carry):
        pltpu.make_async_copy(rows_v, acc.at[pl.ds(r0 + k * step, step)],
                              sem).wait()
        return carry

    lax.fori_loop(0, _ZROWS // step, zwait, 0)


def _flush_acc(rows_v, acc, r0, cid, out_hbm, step):
    def fbody(k, carry):
        rr = r0 + k * step
        pltpu.sync_copy(acc.at[pl.ds(rr, step)], rows_v)
        pltpu.sync_copy(rows_v, out_hbm.at[cid, pl.ds(rr, step)])
        return carry

    lax.fori_loop(0, _ZROWS // step, fbody, 0)


_DEPTH = 4
_GCH = 64
_TSC = _NCHP * _CHUNK // _GCH // _NW
_PH = _TSC // 2


def _sc_agg_body(h_hbm, src_hbm, dst_hbm, z_hbm, out_hbm, *scr):
    src2_v = scr[0]
    dsts = scr[1:1 + _DEPTH]
    rows = scr[1 + _DEPTH:1 + 2 * _DEPTH]
    acc = scr[1 + 2 * _DEPTH]
    gsems = scr[2 + 2 * _DEPTH:2 + 3 * _DEPTH]
    dsems = scr[2 + 3 * _DEPTH:2 + 4 * _DEPTH]
    cid = lax.axis_index("c")
    sid = lax.axis_index("s")
    wid = sid * _NC + cid
    r0 = sid * _ZROWS
    _zero_acc(rows[0], acc, r0, z_hbm, _GCH, gsems[0])

    c0 = wid * _TSC
    pltpu.sync_copy(src_hbm.at[pl.ds(c0, _PH)], src2_v)
    plsc.subcore_barrier()

    def run_pipe(gb):
        def dst_ref(j):
            return dst_hbm.at[pl.ds((c0 + gb + j) * _GCH, _GCH)]

        def fire(j, p):
            pltpu.async_copy(dst_ref(j), dsts[p], dsems[p])
            pltpu.async_copy(h_hbm.at[src2_v.at[j]], rows[p], gsems[p])

        for p in range(_DEPTH):
            fire(p, p)

        def body(t, carry):
            for p in range(_DEPTH):
                j = _DEPTH * t + p
                pltpu.make_async_copy(dst_ref(j), dsts[p], dsems[p]).wait()
                pltpu.make_async_copy(h_hbm.at[src2_v.at[j]], rows[p],
                                      gsems[p]).wait()
                pltpu.sync_copy(rows[p], acc.at[dsts[p]], add=True)

                @pl.when(t < _PH // _DEPTH - 1)
                def _():
                    fire(j + _DEPTH, p)
            return carry

        lax.fori_loop(0, _PH // _DEPTH, body, 0)

    run_pipe(0)
    pltpu.sync_copy(src_hbm.at[pl.ds(c0 + _PH, _PH)], src2_v)
    run_pipe(_PH)

    plsc.subcore_barrier()
    _flush_acc(rows[0], acc, r0, cid, out_hbm, _GCH)


def _sc_deg_body(dst_hbm, z_hbm, ones_hbm, out_hbm,
                 dst_v, rows_v, acc, sem):
    cid = lax.axis_index("c")
    sid = lax.axis_index("s")
    wid = sid * _NC + cid
    r0 = sid * _ZROWS
    _zero_acc(rows_v, acc, r0, z_hbm, _CHUNK, sem)
    pltpu.sync_copy(ones_hbm, rows_v)
    plsc.subcore_barrier()

    e0 = wid * _TCH * _CHUNK

    def body(j, carry):
        pltpu.sync_copy(dst_hbm.at[pl.ds(e0 + j * _CHUNK, _CHUNK)], dst_v)
        pltpu.sync_copy(rows_v, acc.at[dst_v], add=True)
        return carry

    lax.fori_loop(0, _TCH, body, 0)

    plsc.subcore_barrier()
    _flush_acc(rows_v, acc, r0, cid, out_hbm, _CHUNK)


_PLANES = jax.ShapeDtypeStruct((_NC, _NP, _F), jnp.float32)


@functools.lru_cache(maxsize=None)
def _make_sc_agg():
    mesh = plsc.VectorSubcoreMesh(core_axis_name="c", subcore_axis_name="s",
                                  num_cores=_NC, num_subcores=_NS)
    return pl.kernel(
        _sc_agg_body,
        out_type=_PLANES,
        mesh=mesh,
        scratch_types=(
            [pltpu.VMEM((_PH, _GCH), jnp.int32)]
            + [pltpu.VMEM((_GCH,), jnp.int32)] * _DEPTH
            + [pltpu.VMEM((_GCH, _F), jnp.float32)] * _DEPTH
            + [pltpu.VMEM_SHARED((_NP, _F), jnp.float32)]
            + [pltpu.SemaphoreType.DMA] * (2 * _DEPTH)
        ),
    )


@functools.lru_cache(maxsize=None)
def _make_sc_deg():
    mesh = plsc.VectorSubcoreMesh(core_axis_name="c", subcore_axis_name="s",
                                  num_cores=_NC, num_subcores=_NS)
    return pl.kernel(
        _sc_deg_body,
        out_type=_PLANES,
        mesh=mesh,
        scratch_types=[
            pltpu.VMEM((_CHUNK,), jnp.int32),
            pltpu.VMEM((_CHUNK, _F), jnp.float32),
            pltpu.VMEM_SHARED((_NP, _F), jnp.float32),
            pltpu.SemaphoreType.DMA,
        ],
    )


def _deg_body(d_ref, o_ref):
    o_ref[...] = 1.0 / jnp.maximum(d_ref[0] + d_ref[1], 1.0)


_deg_prep = pl.pallas_call(
    _deg_body,
    grid=(_NP // 2048,),
    in_specs=[pl.BlockSpec((_NC, 2048, _F), lambda r: (0, r, 0))],
    out_specs=pl.BlockSpec((2048, _F), lambda r: (r, 0)),
    out_shape=jax.ShapeDtypeStruct((_NP, _F), jnp.float32),
)

_BLK = 2048


def _layer_body(i, act, eps_ref, h_ref, p_ref, inv_ref, w_ref, b_ref, o_ref):
    agg = (p_ref[0] + p_ref[1]) * inv_ref[...]
    rst = (1.0 + eps_ref[i]) * h_ref[...] + agg
    o = jnp.dot(rst, w_ref[...], preferred_element_type=jnp.float32) + b_ref[...]
    o_ref[...] = jnp.maximum(o, 0.0) if act else o


def _make_layer(i, act):
    return pl.pallas_call(
        functools.partial(_layer_body, i, act),
        grid=(_NP // _BLK,),
        in_specs=[
            pl.BlockSpec(memory_space=pltpu.SMEM),
            pl.BlockSpec((_BLK, _F), lambda r: (r, 0)),
            pl.BlockSpec((_NC, _BLK, _F), lambda r: (0, r, 0)),
            pl.BlockSpec((_BLK, _F), lambda r: (r, 0)),
            pl.BlockSpec((_F, _F), lambda r: (0, 0)),
            pl.BlockSpec((1, _F), lambda r: (0, 0)),
        ],
        out_specs=pl.BlockSpec((_BLK, _F), lambda r: (r, 0)),
        out_shape=jax.ShapeDtypeStruct((_NP, _F), jnp.float32),
    )


_layers = [_make_layer(0, True), _make_layer(1, True),
           _make_layer(2, True), _make_layer(3, False)]


def kernel(features, edge_index, W0, b0, W1, b1, W2, b2, W3, b3, eps):
    h = jnp.zeros((_NP, _F), jnp.float32).at[:_N].set(features)
    src_pad = jnp.zeros((_EPAD,), jnp.int32)
    dst_pad = _N + (jnp.arange(_EPAD, dtype=jnp.int32) % (_NP - _N))
    src1 = jnp.concatenate([edge_index[0], src_pad])
    dst1 = jnp.concatenate([edge_index[1], dst_pad])
    zeros = jnp.zeros((_CHUNK, _F), jnp.float32)
    ones = jnp.ones((_CHUNK, _F), jnp.float32)

    W3p = jnp.pad(W3, ((0, 0), (0, _F - _NCLS)))
    b3p = jnp.pad(b3, (0, _F - _NCLS))
    ws = [(W0, b0), (W1, b1), (W2, b2), (W3p, b3p)]

    src2d = src1.reshape(-1, _GCH)
    zeros64 = jnp.zeros((_GCH, _F), jnp.float32)
    dplanes = _make_sc_deg()(dst1, zeros, ones)
    inv = _deg_prep(dplanes)
    for i in range(4):
        planes = _make_sc_agg()(h, src2d, dst1, zeros64)
        W, b = ws[i]
        h = _layers[i](eps, h, planes, inv, W, b.reshape(1, _F))
    return h[:_N, :_NCLS]
